# baseline (speedup 1.0000x reference)
import functools

import jax
import jax.numpy as jnp
import numpy as np
from jax import lax
from jax.experimental import pallas as pl
from jax.experimental.pallas import tpu as pltpu
from jax.experimental.pallas import tpu_sc as plsc


def _upsample_matrix() -> np.ndarray:
    W = np.zeros((96, 48), np.float32)
    for j in range(96):
        c = j / 2 - 0.25
        k0 = int(np.floor(c))
        w1 = c - k0
        taps = [(k0, 1.0 - w1), (k0 + 1, w1)]
        valid = [(k, w) for k, w in taps if 0 <= k < 48]
        s = sum(w for _, w in valid)
        for k, w in valid:
            W[j, k] = w / s
    return W


_W96x48 = _upsample_matrix()

_SITES = 8 * 48 * 48
_NC, _NS = 2, 16
_NW = _NC * _NS
_SPW = _SITES // _NW
_CH = 8
_NCHUNK = _SPW // _CH

_mesh = plsc.VectorSubcoreMesh(core_axis_name="c", subcore_axis_name="s")


@functools.partial(
    pl.kernel,
    out_type=jax.ShapeDtypeStruct((_SITES // 8, 128), jnp.float32),
    mesh=_mesh,
    scratch_types=[
        pltpu.VMEM((_CH, 48, 48), jnp.float32),
        pltpu.VMEM((_CH, 48, 48), jnp.float32),
        pltpu.VMEM((_SPW // 8, 128), jnp.float32),
        pltpu.SemaphoreType.DMA,
        pltpu.SemaphoreType.DMA,
    ],
)
def _sc_mean(vol_hbm, out_hbm, buf0, buf1, means_v, sem0, sem1):
    wid = lax.axis_index("s") * _NC + lax.axis_index("c")
    base = wid * _SPW

    def chunk_src(c):
        site = base + c * _CH
        b = site // (48 * 48)
        h1 = (site // 48) % 48
        w0 = site % 48
        return vol_hbm.at[b, h1, pl.ds(w0, _CH)]

    def compute(buf, c):
        @pl.loop(0, _CH)
        def site_body(s):
            acc = jnp.zeros((16,), jnp.float32)
            for r in range(48):
                for k in range(3):
                    acc = acc + buf[s, r, pl.ds(k * 16, 16)]
            idx = c * _CH + s
            means_v[idx // 8, pl.ds((idx % 8) * 16, 16)] = acc * (1.0 / 2304.0)

    pltpu.async_copy(chunk_src(0), buf0, sem0)
    pltpu.async_copy(chunk_src(1), buf1, sem1)

    @pl.loop(0, _NCHUNK, step=2)
    def chunk_body(c):
        pltpu.make_async_copy(chunk_src(c), buf0, sem0).wait()
        compute(buf0, c)

        @pl.when(c + 2 < _NCHUNK)
        def _():
            pltpu.async_copy(chunk_src(c + 2), buf0, sem0)

        pltpu.make_async_copy(chunk_src(c + 1), buf1, sem1).wait()
        compute(buf1, c + 1)

        @pl.when(c + 3 < _NCHUNK)
        def _():
            pltpu.async_copy(chunk_src(c + 3), buf1, sem1)

    pltpu.sync_copy(means_v, out_hbm.at[pl.ds(wid * (_SPW // 8), _SPW // 8)])


def _head_body(m16_ref, w_ref, map_ref, vol_ref, loss_ref):
    w = w_ref[...]
    x = m16_ref[...]
    m_all = jnp.sum(x.reshape(_SITES // 8, 8, 16), axis=-1).reshape(8, 48, 48)
    acc = jnp.float32(0.0)
    for b in range(8):
        m_b = m_all[b]
        t = jax.lax.dot(w, m_b, precision=jax.lax.Precision.HIGHEST)
        up = jax.lax.dot_general(
            t, w, (((1,), (1,)), ((), ())),
            precision=jax.lax.Precision.HIGHEST)
        vol_ref[b] = up
        d = up - map_ref[b]
        acc = acc + jnp.sum(d * d)
    loss_ref[0, 0] = acc * (1.0 / (8 * 96 * 96))


@jax.jit
def kernel(out_volume, out_map, label):
    del label

    m = _sc_mean(out_volume)

    wmat = jnp.asarray(_W96x48)
    map3 = out_map.reshape(8, 96, 96)

    out_vol, loss = pl.pallas_call(
        _head_body,
        in_specs=[
            pl.BlockSpec((_SITES // 8, 128), lambda: (0, 0)),
            pl.BlockSpec((96, 48), lambda: (0, 0)),
            pl.BlockSpec((8, 96, 96), lambda: (0, 0, 0)),
        ],
        out_specs=[
            pl.BlockSpec((8, 96, 96), lambda: (0, 0, 0)),
            pl.BlockSpec(memory_space=pltpu.SMEM),
        ],
        out_shape=[
            jax.ShapeDtypeStruct((8, 96, 96), jnp.float32),
            jax.ShapeDtypeStruct((1, 1), jnp.float32),
        ],
    )(m, wmat, map3)

    return loss[0, 0], out_vol


# speedup vs baseline: 1.2881x; 1.2881x over previous
import functools

import jax
import jax.numpy as jnp
import numpy as np
from jax import lax
from jax.experimental import pallas as pl
from jax.experimental.pallas import tpu as pltpu
from jax.experimental.pallas import tpu_sc as plsc


def _upsample_matrix() -> np.ndarray:
    W = np.zeros((96, 48), np.float32)
    for j in range(96):
        c = j / 2 - 0.25
        k0 = int(np.floor(c))
        w1 = c - k0
        taps = [(k0, 1.0 - w1), (k0 + 1, w1)]
        valid = [(k, w) for k, w in taps if 0 <= k < 48]
        s = sum(w for _, w in valid)
        for k, w in valid:
            W[j, k] = w / s
    return W


_W96x48 = _upsample_matrix()

_H_SC = 16
_SC_SITES = 8 * _H_SC * 48
_NC, _NS = 2, 16
_NW = _NC * _NS
_SPW = _SC_SITES // _NW
_CH = 8
_NCHUNK = _SPW // _CH

_BH1 = 8

_mesh = plsc.VectorSubcoreMesh(core_axis_name="c", subcore_axis_name="s")


@functools.partial(
    pl.kernel,
    out_type=jax.ShapeDtypeStruct((_SC_SITES // 8, 128), jnp.float32),
    mesh=_mesh,
    scratch_types=[
        pltpu.VMEM((_CH, 48, 48), jnp.float32),
        pltpu.VMEM((_CH, 48, 48), jnp.float32),
        pltpu.VMEM((_SPW // 8, 128), jnp.float32),
        pltpu.SemaphoreType.DMA,
        pltpu.SemaphoreType.DMA,
    ],
)
def _sc_mean(vol_hbm, out_hbm, buf0, buf1, means_v, sem0, sem1):
    wid = lax.axis_index("s") * _NC + lax.axis_index("c")
    base = wid * _SPW

    def chunk_src(c):
        site = base + c * _CH
        b = site // (_H_SC * 48)
        h1 = (site // 48) % _H_SC
        w0 = site % 48
        return vol_hbm.at[b, h1, pl.ds(w0, _CH)]

    def compute(buf, c):
        @pl.loop(0, _CH)
        def site_body(s):
            acc = jnp.zeros((16,), jnp.float32)
            for r in range(48):
                for k in range(3):
                    acc = acc + buf[s, r, pl.ds(k * 16, 16)]
            idx = c * _CH + s
            means_v[idx // 8, pl.ds((idx % 8) * 16, 16)] = acc * (1.0 / 2304.0)

    pltpu.async_copy(chunk_src(0), buf0, sem0)
    pltpu.async_copy(chunk_src(1), buf1, sem1)

    @pl.loop(0, _NCHUNK, step=2)
    def chunk_body(c):
        pltpu.make_async_copy(chunk_src(c), buf0, sem0).wait()
        compute(buf0, c)

        @pl.when(c + 2 < _NCHUNK)
        def _():
            pltpu.async_copy(chunk_src(c + 2), buf0, sem0)

        pltpu.make_async_copy(chunk_src(c + 1), buf1, sem1).wait()
        compute(buf1, c + 1)

        @pl.when(c + 3 < _NCHUNK)
        def _():
            pltpu.async_copy(chunk_src(c + 3), buf1, sem1)

    pltpu.sync_copy(means_v, out_hbm.at[pl.ds(wid * (_SPW // 8), _SPW // 8)])


def _tc_mean_body(vol_ref, mean_ref):
    s = jnp.sum(vol_ref[0], axis=(-2, -1)) * (1.0 / 2304.0)
    mean_ref[...] = s[None]


def _head_body(m16_ref, mtc_ref, w_ref, map_ref, vol_ref, loss_ref):
    w = w_ref[...]
    x = m16_ref[...]
    m_sc = jnp.sum(x.reshape(_SC_SITES // 8, 8, 16), axis=-1)
    m_sc = m_sc.reshape(8, _H_SC, 48)
    m_all = jnp.concatenate([m_sc, mtc_ref[...]], axis=1)
    acc = jnp.float32(0.0)
    for b in range(8):
        t = jax.lax.dot(w, m_all[b], precision=jax.lax.Precision.HIGHEST)
        up = jax.lax.dot_general(
            t, w, (((1,), (1,)), ((), ())),
            precision=jax.lax.Precision.HIGHEST)
        vol_ref[b] = up
        d = up - map_ref[b]
        acc = acc + jnp.sum(d * d)
    loss_ref[0, 0] = acc * (1.0 / (8 * 96 * 96))


@jax.jit
def kernel(out_volume, out_map, label):
    del label

    m16 = _sc_mean(out_volume)

    n_tc = 48 - _H_SC
    m_tc = pl.pallas_call(
        _tc_mean_body,
        grid=(8, n_tc // _BH1),
        in_specs=[pl.BlockSpec((1, _BH1, 48, 48, 48),
                               lambda b, i: (b, i + _H_SC // _BH1, 0, 0, 0))],
        out_specs=pl.BlockSpec((1, _BH1, 48), lambda b, i: (b, i, 0)),
        out_shape=jax.ShapeDtypeStruct((8, n_tc, 48), jnp.float32),
    )(out_volume)

    wmat = jnp.asarray(_W96x48)
    map3 = out_map.reshape(8, 96, 96)

    out_vol, loss = pl.pallas_call(
        _head_body,
        in_specs=[
            pl.BlockSpec((_SC_SITES // 8, 128), lambda: (0, 0)),
            pl.BlockSpec((8, n_tc, 48), lambda: (0, 0, 0)),
            pl.BlockSpec((96, 48), lambda: (0, 0)),
            pl.BlockSpec((8, 96, 96), lambda: (0, 0, 0)),
        ],
        out_specs=[
            pl.BlockSpec((8, 96, 96), lambda: (0, 0, 0)),
            pl.BlockSpec(memory_space=pltpu.SMEM),
        ],
        out_shape=[
            jax.ShapeDtypeStruct((8, 96, 96), jnp.float32),
            jax.ShapeDtypeStruct((1, 1), jnp.float32),
        ],
    )(m16, m_tc, wmat, map3)

    return loss[0, 0], out_vol


# speedup vs baseline: 1.5109x; 1.1730x over previous
import functools

import jax
import jax.numpy as jnp
import numpy as np
from jax.experimental import pallas as pl
from jax.experimental.pallas import tpu as pltpu


def _upsample_matrix() -> np.ndarray:
    W = np.zeros((96, 48), np.float32)
    for j in range(96):
        c = j / 2 - 0.25
        k0 = int(np.floor(c))
        w1 = c - k0
        taps = [(k0, 1.0 - w1), (k0 + 1, w1)]
        valid = [(k, w) for k, w in taps if 0 <= k < 48]
        s = sum(w for _, w in valid)
        for k, w in valid:
            W[j, k] = w / s
    return W


_W96x48 = _upsample_matrix()

_BH1 = 8
_NI = 48 // _BH1


def _fused_body(vol_ref, w_ref, map_ref, vol_out_ref, loss_ref, m_s, acc_s):
    b = pl.program_id(0)
    i = pl.program_id(1)

    sums = jnp.sum(vol_ref[0], axis=(-2, -1)) * (1.0 / 2304.0)
    m_s[pl.ds(i * _BH1, _BH1), :] = sums

    @pl.when(jnp.logical_and(b == 0, i == 0))
    def _():
        acc_s[0] = jnp.float32(0.0)

    @pl.when(i == _NI - 1)
    def _():
        w = w_ref[...]
        t = jax.lax.dot(w, m_s[...], precision=jax.lax.Precision.HIGHEST)
        up = jax.lax.dot_general(
            t, w, (((1,), (1,)), ((), ())),
            precision=jax.lax.Precision.HIGHEST)
        vol_out_ref[0] = up
        d = up - map_ref[0]
        acc_s[0] = acc_s[0] + jnp.sum(d * d)

        @pl.when(b == 7)
        def _():
            loss_ref[0, 0] = acc_s[0] * (1.0 / (8 * 96 * 96))


@jax.jit
def kernel(out_volume, out_map, label):
    del label

    wmat = jnp.asarray(_W96x48)
    map3 = out_map.reshape(8, 96, 96)

    out_vol, loss = pl.pallas_call(
        _fused_body,
        grid=(8, _NI),
        in_specs=[
            pl.BlockSpec((1, _BH1, 48, 48, 48),
                         lambda b, i: (b, i, 0, 0, 0)),
            pl.BlockSpec((96, 48), lambda b, i: (0, 0)),
            pl.BlockSpec((1, 96, 96), lambda b, i: (b, 0, 0)),
        ],
        out_specs=[
            pl.BlockSpec((1, 96, 96), lambda b, i: (b, 0, 0)),
            pl.BlockSpec(memory_space=pltpu.SMEM),
        ],
        out_shape=[
            jax.ShapeDtypeStruct((8, 96, 96), jnp.float32),
            jax.ShapeDtypeStruct((1, 1), jnp.float32),
        ],
        scratch_shapes=[
            pltpu.VMEM((48, 48), jnp.float32),
            pltpu.SMEM((1,), jnp.float32),
        ],
    )(out_volume, wmat, map3)

    return loss[0, 0], out_vol


# speedup vs baseline: 1.5148x; 1.0026x over previous
import functools

import jax
import jax.numpy as jnp
import numpy as np
from jax.experimental import pallas as pl
from jax.experimental.pallas import tpu as pltpu


def _upsample_matrix() -> np.ndarray:
    W = np.zeros((96, 48), np.float32)
    for j in range(96):
        c = j / 2 - 0.25
        k0 = int(np.floor(c))
        w1 = c - k0
        taps = [(k0, 1.0 - w1), (k0 + 1, w1)]
        valid = [(k, w) for k, w in taps if 0 <= k < 48]
        s = sum(w for _, w in valid)
        for k, w in valid:
            W[j, k] = w / s
    return W


_W96x48 = _upsample_matrix()

_BH1 = 8
_NI = 48 // _BH1


def _fused_body(v1_ref, v2_ref, w_ref, map_ref, vol_out_ref, loss_ref,
                m_s, acc_s):
    b = pl.program_id(0)
    i = pl.program_id(1)

    sums = (jnp.sum(v1_ref[0], axis=(-2, -1))
            + jnp.sum(v2_ref[0], axis=(-2, -1))) * (1.0 / 2304.0)
    m_s[pl.ds(i * _BH1, _BH1), :] = sums

    @pl.when(jnp.logical_and(b == 0, i == 0))
    def _():
        acc_s[0] = jnp.float32(0.0)

    @pl.when(i == _NI - 1)
    def _():
        w = w_ref[...]
        t = jax.lax.dot(w, m_s[...], precision=jax.lax.Precision.HIGHEST)
        up = jax.lax.dot_general(
            t, w, (((1,), (1,)), ((), ())),
            precision=jax.lax.Precision.HIGHEST)
        vol_out_ref[0] = up
        d = up - map_ref[0]
        acc_s[0] = acc_s[0] + jnp.sum(d * d)

        @pl.when(b == 7)
        def _():
            loss_ref[0, 0] = acc_s[0] * (1.0 / (8 * 96 * 96))


@jax.jit
def kernel(out_volume, out_map, label):
    del label

    wmat = jnp.asarray(_W96x48)
    map3 = out_map.reshape(8, 96, 96)

    out_vol, loss = pl.pallas_call(
        _fused_body,
        grid=(8, _NI),
        in_specs=[
            pl.BlockSpec((1, _BH1, 48, 24, 48),
                         lambda b, i: (b, i, 0, 0, 0)),
            pl.BlockSpec((1, _BH1, 48, 24, 48),
                         lambda b, i: (b, i, 0, 1, 0)),
            pl.BlockSpec((96, 48), lambda b, i: (0, 0)),
            pl.BlockSpec((1, 96, 96), lambda b, i: (b, 0, 0)),
        ],
        out_specs=[
            pl.BlockSpec((1, 96, 96), lambda b, i: (b, 0, 0)),
            pl.BlockSpec(memory_space=pltpu.SMEM),
        ],
        out_shape=[
            jax.ShapeDtypeStruct((8, 96, 96), jnp.float32),
            jax.ShapeDtypeStruct((1, 1), jnp.float32),
        ],
        scratch_shapes=[
            pltpu.VMEM((48, 48), jnp.float32),
            pltpu.SMEM((1,), jnp.float32),
        ],
    )(out_volume, out_volume, wmat, map3)

    return loss[0, 0], out_vol
